# baseline (speedup 1.0000x reference)
import jax
import jax.numpy as jnp
from jax import lax
from jax.experimental import pallas as pl
from jax.experimental.pallas import tpu as pltpu
from jax.experimental.pallas import tpu_sc as plsc

B = 64
S = 4096
D = 1024
KV = 32

NLANE = 16
NSEG = 16
SEGLEN = S // NSEG
NCHUNK = SEGLEN // NLANE

NC = 2
NS = 16
NW = NC * NS
ROWS_PER_W = B // NW

_INT_MAX = 2**31 - 1
_NEG_INF = float("-inf")


def _dyn_gather(x, idx):
    return lax.gather(
        x, idx[:, None],
        dimension_numbers=lax.GatherDimensionNumbers(
            offset_dims=(), collapsed_slice_dims=(0,), start_index_map=(0,)),
        slice_sizes=(1,),
        mode=lax.GatherScatterMode.PROMISE_IN_BOUNDS,
    )


def _tec_body(att_hbm, ctx_hbm, out_hbm, att0_v, att1_v, rows0_v, rows1_v,
              s_a0, s_a1, s_g0, s_g1, s_o0, s_o1):
    wid = lax.axis_index("s") * NC + lax.axis_index("c")
    lanes = lax.iota(jnp.int32, NLANE)

    def allmax(v):
        for sh in (8, 4, 2, 1):
            v = jnp.maximum(v, _dyn_gather(v, jnp.bitwise_xor(lanes, sh)))
        return v

    def allmin(v):
        for sh in (8, 4, 2, 1):
            v = jnp.minimum(v, _dyn_gather(v, jnp.bitwise_xor(lanes, sh)))
        return v

    def seg_scan(att_v, seg_base):
        m = att_v[pl.ds(seg_base, NLANE)]
        g = seg_base + lanes
        for j in range(1, NCHUNK):
            base = seg_base + j * NLANE
            v = att_v[pl.ds(base, NLANE)]
            upd = v > m
            m = jnp.where(upd, v, m)
            g = jnp.where(upd, base + lanes, g)
        smax = allmax(m)
        sarg = allmin(jnp.where(m == smax, g, _INT_MAX))
        return smax, sarg

    def build_one(att_v, s, segmax, segarg):
        smax, sarg = seg_scan(att_v, s * SEGLEN)
        lane_is_s = lanes == s
        return (jnp.where(lane_is_s, smax, segmax),
                jnp.where(lane_is_s, sarg, segarg))

    def extract_one(att_v, segmax, segarg):
        gmax = allmax(segmax)
        g_vec = allmin(jnp.where(segmax == gmax, segarg, _INT_MAX))
        g = g_vec[0]
        lane = jnp.bitwise_and(g, NLANE - 1)
        cbase = g - lane
        v = att_v[pl.ds(cbase, NLANE)]
        att_v[pl.ds(cbase, NLANE)] = jnp.where(lanes == lane, _NEG_INF, v)
        s_star = lax.shift_right_logical(g, 8)
        smax, sarg = seg_scan(att_v, s_star * SEGLEN)
        lane_is_s = lanes == s_star
        return (g_vec,
                jnp.where(lane_is_s, smax, segmax),
                jnp.where(lane_is_s, sarg, segarg))

    b0 = wid * ROWS_PER_W
    b1 = b0 + 1

    a0 = pltpu.async_copy(att_hbm.at[b0], att0_v, s_a0)
    a1 = pltpu.async_copy(att_hbm.at[b1], att1_v, s_a1)
    a0.wait()
    a1.wait()

    def build(s, carry):
        sm0, sa0, sm1, sa1 = carry
        sm0, sa0 = build_one(att0_v, s, sm0, sa0)
        sm1, sa1 = build_one(att1_v, s, sm1, sa1)
        return (sm0, sa0, sm1, sa1)

    neg = jnp.full((NLANE,), _NEG_INF, jnp.float32)
    zer = jnp.zeros((NLANE,), jnp.int32)
    sm0, sa0, sm1, sa1 = lax.fori_loop(0, NSEG, build, (neg, zer, neg, zer))

    def extract_lo(k, carry):
        sm0, sa0, sm1, sa1, lo0, lo1 = carry
        g0_vec, sm0, sa0 = extract_one(att0_v, sm0, sa0)
        g1_vec, sm1, sa1 = extract_one(att1_v, sm1, sa1)
        lane_is_k = lanes == k
        lo0 = jnp.where(lane_is_k, b0 * S + g0_vec, lo0)
        lo1 = jnp.where(lane_is_k, b1 * S + g1_vec, lo1)
        return (sm0, sa0, sm1, sa1, lo0, lo1)

    sm0, sa0, sm1, sa1, lo0, lo1 = lax.fori_loop(
        0, NLANE, extract_lo, (sm0, sa0, sm1, sa1, zer, zer))

    g0a = pltpu.async_copy(ctx_hbm.at[lo0], rows0_v.at[pl.ds(0, NLANE)], s_g0)
    g1a = pltpu.async_copy(ctx_hbm.at[lo1], rows1_v.at[pl.ds(0, NLANE)], s_g1)

    def extract_hi(k, carry):
        sm0, sa0, sm1, sa1, hi0, hi1 = carry
        g0_vec, sm0, sa0 = extract_one(att0_v, sm0, sa0)
        g1_vec, sm1, sa1 = extract_one(att1_v, sm1, sa1)
        lane_is_k = lanes == k
        hi0 = jnp.where(lane_is_k, b0 * S + g0_vec, hi0)
        hi1 = jnp.where(lane_is_k, b1 * S + g1_vec, hi1)
        return (sm0, sa0, sm1, sa1, hi0, hi1)

    _, _, _, _, hi0, hi1 = lax.fori_loop(
        0, NLANE, extract_hi, (sm0, sa0, sm1, sa1, zer, zer))

    g0b = pltpu.async_copy(ctx_hbm.at[hi0], rows0_v.at[pl.ds(NLANE, NLANE)],
                           s_g0)
    g1b = pltpu.async_copy(ctx_hbm.at[hi1], rows1_v.at[pl.ds(NLANE, NLANE)],
                           s_g1)

    g0a.wait()
    o0a = pltpu.async_copy(rows0_v.at[pl.ds(0, NLANE)],
                           out_hbm.at[b0, pl.ds(0, NLANE)], s_o0)
    g1a.wait()
    o1a = pltpu.async_copy(rows1_v.at[pl.ds(0, NLANE)],
                           out_hbm.at[b1, pl.ds(0, NLANE)], s_o1)
    g0b.wait()
    o0b = pltpu.async_copy(rows0_v.at[pl.ds(NLANE, NLANE)],
                           out_hbm.at[b0, pl.ds(NLANE, NLANE)], s_o0)
    g1b.wait()
    o1b = pltpu.async_copy(rows1_v.at[pl.ds(NLANE, NLANE)],
                           out_hbm.at[b1, pl.ds(NLANE, NLANE)], s_o1)
    o0a.wait()
    o1a.wait()
    o0b.wait()
    o1b.wait()


_sc_call = pl.kernel(
    _tec_body,
    out_type=jax.ShapeDtypeStruct((B, KV, D), jnp.float32),
    mesh=plsc.VectorSubcoreMesh(core_axis_name="c", subcore_axis_name="s"),
    scratch_types=[
        pltpu.VMEM((S,), jnp.float32),
        pltpu.VMEM((S,), jnp.float32),
        pltpu.VMEM((KV, D), jnp.float32),
        pltpu.VMEM((KV, D), jnp.float32),
        pltpu.SemaphoreType.DMA,
        pltpu.SemaphoreType.DMA,
        pltpu.SemaphoreType.DMA,
        pltpu.SemaphoreType.DMA,
        pltpu.SemaphoreType.DMA,
        pltpu.SemaphoreType.DMA,
    ],
)


def kernel(attention, context, K):
    del K
    return _sc_call(attention, context.reshape(B * S, D))


# speedup vs baseline: 1.0078x; 1.0078x over previous
import jax
import jax.numpy as jnp
from jax import lax
from jax.experimental import pallas as pl
from jax.experimental.pallas import tpu as pltpu
from jax.experimental.pallas import tpu_sc as plsc

B = 64
S = 4096
D = 1024
KV = 32

NLANE = 16
NSEG = 32
SEGLEN = S // NSEG
SEGSHIFT = 7
NCHUNK = SEGLEN // NLANE

NC = 2
NS = 16
NW = NC * NS
ROWS_PER_W = B // NW

_INT_MAX = 2**31 - 1
_NEG_INF = float("-inf")


def _dyn_gather(x, idx):
    return lax.gather(
        x, idx[:, None],
        dimension_numbers=lax.GatherDimensionNumbers(
            offset_dims=(), collapsed_slice_dims=(0,), start_index_map=(0,)),
        slice_sizes=(1,),
        mode=lax.GatherScatterMode.PROMISE_IN_BOUNDS,
    )


def _tec_body(att_hbm, ctx_hbm, out_hbm, att0_v, att1_v, rows0_v, rows1_v,
              s_a0, s_a1, s_g0, s_g1, s_o0, s_o1):
    wid = lax.axis_index("s") * NC + lax.axis_index("c")
    lanes = lax.iota(jnp.int32, NLANE)

    def allmax(v):
        for sh in (8, 4, 2, 1):
            v = jnp.maximum(v, _dyn_gather(v, jnp.bitwise_xor(lanes, sh)))
        return v

    def allmin(v):
        for sh in (8, 4, 2, 1):
            v = jnp.minimum(v, _dyn_gather(v, jnp.bitwise_xor(lanes, sh)))
        return v

    def seg_scan(att_v, seg_base):
        m = att_v[pl.ds(seg_base, NLANE)]
        g = seg_base + lanes
        for j in range(1, NCHUNK):
            base = seg_base + j * NLANE
            v = att_v[pl.ds(base, NLANE)]
            upd = v > m
            m = jnp.where(upd, v, m)
            g = jnp.where(upd, base + lanes, g)
        smax = allmax(m)
        sarg = allmin(jnp.where(m == smax, g, _INT_MAX))
        return smax, sarg

    def extract_one(att_v, smA, saA, smB, saB):
        gmax = allmax(jnp.maximum(smA, smB))
        candA = jnp.where(smA == gmax, saA, _INT_MAX)
        candB = jnp.where(smB == gmax, saB, _INT_MAX)
        g_vec = allmin(jnp.minimum(candA, candB))
        g = g_vec[0]
        lane = jnp.bitwise_and(g, NLANE - 1)
        cbase = g - lane
        v = att_v[pl.ds(cbase, NLANE)]
        att_v[pl.ds(cbase, NLANE)] = jnp.where(lanes == lane, _NEG_INF, v)
        s_star = lax.shift_right_logical(g, SEGSHIFT)
        smax, sarg = seg_scan(att_v, s_star * SEGLEN)
        upd_a = lanes == s_star
        upd_b = lanes == (s_star - NLANE)
        return (g_vec,
                jnp.where(upd_a, smax, smA), jnp.where(upd_a, sarg, saA),
                jnp.where(upd_b, smax, smB), jnp.where(upd_b, sarg, saB))

    b0 = wid * ROWS_PER_W
    b1 = b0 + 1

    a0 = pltpu.async_copy(att_hbm.at[b0], att0_v, s_a0)
    a1 = pltpu.async_copy(att_hbm.at[b1], att1_v, s_a1)
    a0.wait()
    a1.wait()

    def build_a(s, carry):
        sm0, sa0, sm1, sa1 = carry
        x0, a0_ = seg_scan(att0_v, s * SEGLEN)
        x1, a1_ = seg_scan(att1_v, s * SEGLEN)
        lane_is_s = lanes == s
        return (jnp.where(lane_is_s, x0, sm0), jnp.where(lane_is_s, a0_, sa0),
                jnp.where(lane_is_s, x1, sm1), jnp.where(lane_is_s, a1_, sa1))

    def build_b(s, carry):
        sm0, sa0, sm1, sa1 = carry
        x0, a0_ = seg_scan(att0_v, s * SEGLEN)
        x1, a1_ = seg_scan(att1_v, s * SEGLEN)
        lane_is_s = lanes == (s - NLANE)
        return (jnp.where(lane_is_s, x0, sm0), jnp.where(lane_is_s, a0_, sa0),
                jnp.where(lane_is_s, x1, sm1), jnp.where(lane_is_s, a1_, sa1))

    neg = jnp.full((NLANE,), _NEG_INF, jnp.float32)
    zer = jnp.zeros((NLANE,), jnp.int32)
    smA0, saA0, smA1, saA1 = lax.fori_loop(
        0, NLANE, build_a, (neg, zer, neg, zer))
    smB0, saB0, smB1, saB1 = lax.fori_loop(
        NLANE, NSEG, build_b, (neg, zer, neg, zer))

    def extract_lo(k, carry):
        c0, c1, lo0, lo1 = carry
        g0_vec, *c0 = extract_one(att0_v, *c0)
        g1_vec, *c1 = extract_one(att1_v, *c1)
        lane_is_k = lanes == k
        lo0 = jnp.where(lane_is_k, b0 * S + g0_vec, lo0)
        lo1 = jnp.where(lane_is_k, b1 * S + g1_vec, lo1)
        return (tuple(c0), tuple(c1), lo0, lo1)

    c0 = (smA0, saA0, smB0, saB0)
    c1 = (smA1, saA1, smB1, saB1)
    c0, c1, lo0, lo1 = lax.fori_loop(
        0, NLANE, extract_lo, (c0, c1, zer, zer))

    g0a = pltpu.async_copy(ctx_hbm.at[lo0], rows0_v.at[pl.ds(0, NLANE)], s_g0)
    g1a = pltpu.async_copy(ctx_hbm.at[lo1], rows1_v.at[pl.ds(0, NLANE)], s_g1)

    def extract_hi(k, carry):
        c0, c1, hi0, hi1 = carry
        g0_vec, *c0 = extract_one(att0_v, *c0)
        g1_vec, *c1 = extract_one(att1_v, *c1)
        lane_is_k = lanes == k
        hi0 = jnp.where(lane_is_k, b0 * S + g0_vec, hi0)
        hi1 = jnp.where(lane_is_k, b1 * S + g1_vec, hi1)
        return (tuple(c0), tuple(c1), hi0, hi1)

    _, _, hi0, hi1 = lax.fori_loop(
        0, NLANE, extract_hi, (c0, c1, zer, zer))

    g0b = pltpu.async_copy(ctx_hbm.at[hi0], rows0_v.at[pl.ds(NLANE, NLANE)],
                           s_g0)
    g1b = pltpu.async_copy(ctx_hbm.at[hi1], rows1_v.at[pl.ds(NLANE, NLANE)],
                           s_g1)

    g0a.wait()
    g0b.wait()
    o0 = pltpu.async_copy(rows0_v, out_hbm.at[b0], s_o0)
    g1a.wait()
    g1b.wait()
    o1 = pltpu.async_copy(rows1_v, out_hbm.at[b1], s_o1)
    o0.wait()
    o1.wait()


_sc_call = pl.kernel(
    _tec_body,
    out_type=jax.ShapeDtypeStruct((B, KV, D), jnp.float32),
    mesh=plsc.VectorSubcoreMesh(core_axis_name="c", subcore_axis_name="s"),
    scratch_types=[
        pltpu.VMEM((S,), jnp.float32),
        pltpu.VMEM((S,), jnp.float32),
        pltpu.VMEM((KV, D), jnp.float32),
        pltpu.VMEM((KV, D), jnp.float32),
        pltpu.SemaphoreType.DMA,
        pltpu.SemaphoreType.DMA,
        pltpu.SemaphoreType.DMA,
        pltpu.SemaphoreType.DMA,
        pltpu.SemaphoreType.DMA,
        pltpu.SemaphoreType.DMA,
    ],
)


def kernel(attention, context, K):
    del K
    return _sc_call(attention, context.reshape(B * S, D))
